# Initial kernel scaffold; baseline (speedup 1.0000x reference)
import jax
import jax.numpy as jnp
from jax.experimental import pallas as pl


def kernel(x, edge_small, edge_medium, edge_large, Ws1, bs1, Ws2, bs2, Wm1, bm1, Wm2, bm2, Wl1, bl1, Wl2, bl2, scale_weights):
    raise NotImplementedError("write your pallas kernel here")


# speedup vs baseline: 19.7526x; 19.7526x over previous
import functools

import jax
import jax.numpy as jnp
from jax import lax
from jax.experimental import pallas as pl
from jax.experimental.pallas import tpu as pltpu
from jax.experimental.pallas import tpu_sc as plsc

N = 10000
F = 128
NP = 10240
NC = 2
NS = 16
NW = NC * NS
B = 128

G = 8

E_RAW = (160000, 320000, 640000)
E_PAD = tuple(-(-e // (NW * B * G)) * (NW * B * G) for e in E_RAW)
S32 = tuple(e // (NW * B) for e in E_PAD)
S16 = tuple(e // (NS * B) for e in E_PAD)
S32_MAX = max(S32)
S16_MAX = max(S16)

ROWS_PER_TILE = NP // NS
ROWS_PER_CW = NP // (NC * NS)

@functools.cache
def _mesh():
    return plsc.VectorSubcoreMesh(core_axis_name="c", subcore_axis_name="s")


def _deg_body(dst_s, dst_m, dst_l, zflat, deg_out,
              acc0, acc1, acc2, idxbuf, ones_v, degbuf, rowbuf):
    c = lax.axis_index("c")
    s = lax.axis_index("s")
    accs = (acc0, acc1, acc2)

    for i in range(B // 16):
        ones_v[pl.ds(i * 16, 16)] = jnp.full((16,), 1.0, dtype=jnp.float32)

    for k in range(3):
        pltpu.sync_copy(zflat.at[pl.ds(s * ROWS_PER_TILE, ROWS_PER_TILE)],
                        accs[k].at[pl.ds(s * ROWS_PER_TILE, ROWS_PER_TILE)])
    plsc.subcore_barrier()

    for k, dst3 in enumerate((dst_s, dst_m, dst_l)):
        pltpu.sync_copy(dst3.at[s], idxbuf.at[pl.ds(0, S16[k])])
        acck = accs[k]

        def body(j, _, acck=acck):
            pltpu.sync_copy(ones_v, acck.at[idxbuf.at[j]], add=True)
            return _

        lax.fori_loop(0, S16[k], body, 0)
    plsc.subcore_barrier()

    nbase = (c * NS + s) * ROWS_PER_CW
    for k in range(3):
        pltpu.sync_copy(accs[k].at[pl.ds(nbase, ROWS_PER_CW)], degbuf)

        def rep_body(g, _, k=k, nbase=nbase):
            for i in range(16):
                idxv = lax.broadcast_in_dim(g * 16 + i, (16,), ())
                row = plsc.load_gather(degbuf, [idxv]) + 1.0
                for cc in range(F // 16):
                    rowbuf[i, pl.ds(cc * 16, 16)] = row
            pltpu.sync_copy(rowbuf, deg_out.at[k, pl.ds(nbase + g * 16, 16)])
            return _

        lax.fori_loop(0, ROWS_PER_CW // 16, rep_body, 0)


@functools.cache
def _deg_call():
    return pl.kernel(
        _deg_body,
        out_type=jax.ShapeDtypeStruct((3, NP, F), jnp.float32),
        mesh=_mesh(),
        compiler_params=pltpu.CompilerParams(needs_layout_passes=False),
        scratch_types=[
            pltpu.VMEM_SHARED((NP,), jnp.float32),
            pltpu.VMEM_SHARED((NP,), jnp.float32),
            pltpu.VMEM_SHARED((NP,), jnp.float32),
            pltpu.VMEM((S16_MAX, B), jnp.int32),
            pltpu.VMEM((B,), jnp.float32),
            pltpu.VMEM((ROWS_PER_CW,), jnp.float32),
            pltpu.VMEM((16, F), jnp.float32),
        ],
    )


def _edge_body(tab_s, tab_m, tab_l, src_s, dst_s, src_m, dst_m, src_l, dst_l,
               zrows, pout, acc, srcbuf, dstbuf, rows, sem):
    c = lax.axis_index("c")
    s = lax.axis_index("s")
    w = s * NC + c
    rbase = s * ROWS_PER_TILE

    for k, (tab, src3, dst3) in enumerate(
            ((tab_s, src_s, dst_s), (tab_m, src_m, dst_m), (tab_l, src_l, dst_l))):
        pltpu.sync_copy(zrows.at[pl.ds(rbase, ROWS_PER_TILE)],
                        acc.at[pl.ds(rbase, ROWS_PER_TILE)])
        plsc.subcore_barrier()

        def blk_body(bi, _, tab=tab, src3=src3, dst3=dst3):
            pltpu.sync_copy(src3.at[w, pl.ds(bi * G, G)], srcbuf)
            pltpu.sync_copy(dst3.at[w, pl.ds(bi * G, G)], dstbuf)
            for j in range(G):
                pltpu.async_copy(tab.at[srcbuf.at[j]], rows, sem).wait()
                pltpu.sync_copy(rows, acc.at[dstbuf.at[j]], add=True)
            return _

        lax.fori_loop(0, S32[k] // G, blk_body, 0)
        plsc.subcore_barrier()

        pltpu.sync_copy(acc.at[pl.ds(rbase, ROWS_PER_TILE)],
                        pout.at[c, k, pl.ds(rbase, ROWS_PER_TILE)])
        plsc.subcore_barrier()


@functools.cache
def _edge_call():
    return pl.kernel(
        _edge_body,
        out_type=jax.ShapeDtypeStruct((NC, 3, NP, F), jnp.float32),
        mesh=_mesh(),
        compiler_params=pltpu.CompilerParams(needs_layout_passes=False),
        scratch_types=[
            pltpu.VMEM_SHARED((NP, F), jnp.float32),
            pltpu.VMEM((G, B), jnp.int32),
            pltpu.VMEM((G, B), jnp.int32),
            pltpu.VMEM((B, F), jnp.float32),
            pltpu.SemaphoreType.DMA,
        ],
    )


R = 1024
GRID = NP // R


def _tc1_body(x_ref, ws_ref, wm_ref, wl_ref, deg_ref, hs_ref, hm_ref, hl_ref):
    xb = x_ref[...]
    for k, (w_ref, h_ref) in enumerate(
            ((ws_ref, hs_ref), (wm_ref, hm_ref), (wl_ref, hl_ref))):
        h = jnp.dot(xb, w_ref[...], preferred_element_type=jnp.float32)
        h_ref[...] = h * lax.rsqrt(deg_ref[k])


def _tc2_body(p_ref, hs_ref, hm_ref, hl_ref, deg_ref, b1_ref,
              ws_ref, wm_ref, wl_ref, os_ref, om_ref, ol_ref):
    for k, (h_ref, w_ref, o_ref) in enumerate(
            ((hs_ref, ws_ref, os_ref), (hm_ref, wm_ref, om_ref), (hl_ref, wl_ref, ol_ref))):
        d = lax.rsqrt(deg_ref[k])
        p = p_ref[0, k] + p_ref[1, k] + h_ref[...]
        z = jnp.maximum(d * p + b1_ref[k], 0.0)
        o_ref[...] = jnp.dot(z, w_ref[...], preferred_element_type=jnp.float32) * d


def _tc3_body(p_ref, hs_ref, hm_ref, hl_ref, deg_ref, b2_ref, wmix_ref, out_ref):
    acc = None
    for k, h_ref in enumerate((hs_ref, hm_ref, hl_ref)):
        o = lax.rsqrt(deg_ref[k]) * (p_ref[0, k] + p_ref[1, k] + h_ref[...]) + b2_ref[k]
        t = wmix_ref[k] * o
        acc = t if acc is None else acc + t
    out_ref[...] = acc


def _bs_rows():
    return pl.BlockSpec((R, F), lambda i: (i, 0))


def _bs_w():
    return pl.BlockSpec((F, F), lambda i: (0, 0))


def _bs_dinv():
    return pl.BlockSpec((3, R, F), lambda i: (0, i, 0))


def _bs_part():
    return pl.BlockSpec((NC, 3, R, F), lambda i: (0, 0, i, 0))


def _bs_bias():
    return pl.BlockSpec((3, 1, F), lambda i: (0, 0, 0))


_tc1_call = pl.pallas_call(
    _tc1_body,
    grid=(GRID,),
    in_specs=[_bs_rows(), _bs_w(), _bs_w(), _bs_w(), _bs_dinv()],
    out_specs=[_bs_rows()] * 3,
    out_shape=[jax.ShapeDtypeStruct((NP, F), jnp.float32)] * 3,
)

_tc2_call = pl.pallas_call(
    _tc2_body,
    grid=(GRID,),
    in_specs=[_bs_part(), _bs_rows(), _bs_rows(), _bs_rows(), _bs_dinv(),
              _bs_bias(), _bs_w(), _bs_w(), _bs_w()],
    out_specs=[_bs_rows()] * 3,
    out_shape=[jax.ShapeDtypeStruct((NP, F), jnp.float32)] * 3,
)

_tc3_call = pl.pallas_call(
    _tc3_body,
    grid=(GRID,),
    in_specs=[_bs_part(), _bs_rows(), _bs_rows(), _bs_rows(), _bs_dinv(),
              _bs_bias(), _bs_bias()],
    out_specs=_bs_rows(),
    out_shape=jax.ShapeDtypeStruct((NP, F), jnp.float32),
)


def _pad_edges(edge, k):
    e = E_RAW[k]
    pad = E_PAD[k] - e
    ar = lax.iota(jnp.int32, pad)
    src = jnp.concatenate([edge[0], ar % N])
    dst = jnp.concatenate([edge[1], N + (ar % 16)])
    return src, dst


def kernel(x, edge_small, edge_medium, edge_large, Ws1, bs1, Ws2, bs2,
           Wm1, bm1, Wm2, bm2, Wl1, bl1, Wl2, bl2, scale_weights):
    x_pad = jnp.pad(x, ((0, NP - N), (0, 0)))

    srcs, dsts = [], []
    for k, edge in enumerate((edge_small, edge_medium, edge_large)):
        src, dst = _pad_edges(edge, k)
        srcs.append(src)
        dsts.append(dst)

    deg3 = _deg_call()(dsts[0].reshape(NS, S16[0], B),
                      dsts[1].reshape(NS, S16[1], B),
                      dsts[2].reshape(NS, S16[2], B),
                      jnp.zeros((NP,), jnp.float32))

    h1s, h1m, h1l = _tc1_call(x_pad, Ws1, Wm1, Wl1, deg3)

    zrows = jnp.zeros((NP, F), jnp.float32)
    e32 = []
    for k in range(3):
        e32.append(srcs[k].reshape(NW, S32[k], B))
        e32.append(dsts[k].reshape(NW, S32[k], B))

    p1 = _edge_call()(h1s, h1m, h1l, *e32, zrows)

    b1 = jnp.stack([bs1, bm1, bl1])[:, None, :]
    h2s, h2m, h2l = _tc2_call(p1, h1s, h1m, h1l, deg3, b1, Ws2, Wm2, Wl2)

    p2 = _edge_call()(h2s, h2m, h2l, *e32, zrows)

    b2 = jnp.stack([bs2, bm2, bl2])[:, None, :]
    w = jax.nn.softmax(scale_weights)
    wmix = jnp.broadcast_to(w[:, None, None], (3, 1, F))
    out = _tc3_call(p2, h2s, h2m, h2l, deg3, b2, wmix)
    return out[:N]


# speedup vs baseline: 26.8579x; 1.3597x over previous
import functools

import jax
import jax.numpy as jnp
from jax import lax
from jax.experimental import pallas as pl
from jax.experimental.pallas import tpu as pltpu
from jax.experimental.pallas import tpu_sc as plsc

N = 10000
F = 128
NP = 10240
NC = 2
NS = 16
NW = NC * NS
B = 128

G = 8

E_RAW = (160000, 320000, 640000)
E_PAD = tuple(-(-e // (NW * B * G)) * (NW * B * G) for e in E_RAW)
S32 = tuple(e // (NW * B) for e in E_PAD)
S16 = tuple(e // (NS * B) for e in E_PAD)
S32_MAX = max(S32)
S16_MAX = max(S16)

ROWS_PER_TILE = NP // NS
ROWS_PER_CW = NP // (NC * NS)

@functools.cache
def _mesh():
    return plsc.VectorSubcoreMesh(core_axis_name="c", subcore_axis_name="s")


def _deg_body(dst_s, dst_m, dst_l, zflat, deg_out,
              acc0, acc1, acc2, idxbuf, ones_v, degbuf, rowbuf):
    c = lax.axis_index("c")
    s = lax.axis_index("s")
    accs = (acc0, acc1, acc2)

    for i in range(B // 16):
        ones_v[pl.ds(i * 16, 16)] = jnp.full((16,), 1.0, dtype=jnp.float32)

    for k in range(3):
        pltpu.sync_copy(zflat.at[pl.ds(s * ROWS_PER_TILE, ROWS_PER_TILE)],
                        accs[k].at[pl.ds(s * ROWS_PER_TILE, ROWS_PER_TILE)])
    plsc.subcore_barrier()

    for k, dst3 in enumerate((dst_s, dst_m, dst_l)):
        pltpu.sync_copy(dst3.at[s], idxbuf.at[pl.ds(0, S16[k])])
        acck = accs[k]

        def body(j, _, acck=acck):
            pltpu.sync_copy(ones_v, acck.at[idxbuf.at[j]], add=True)
            return _

        lax.fori_loop(0, S16[k], body, 0)
    plsc.subcore_barrier()

    nbase = (c * NS + s) * ROWS_PER_CW
    for k in range(3):
        pltpu.sync_copy(accs[k].at[pl.ds(nbase, ROWS_PER_CW)], degbuf)

        def rep_body(g, _, k=k, nbase=nbase):
            for i in range(16):
                idxv = lax.broadcast_in_dim(g * 16 + i, (16,), ())
                row = plsc.load_gather(degbuf, [idxv]) + 1.0
                for cc in range(F // 16):
                    rowbuf[i, pl.ds(cc * 16, 16)] = row
            pltpu.sync_copy(rowbuf, deg_out.at[k, pl.ds(nbase + g * 16, 16)])
            return _

        lax.fori_loop(0, ROWS_PER_CW // 16, rep_body, 0)


@functools.cache
def _deg_call():
    return pl.kernel(
        _deg_body,
        out_type=jax.ShapeDtypeStruct((3, NP, F), jnp.float32),
        mesh=_mesh(),
        compiler_params=pltpu.CompilerParams(needs_layout_passes=False),
        scratch_types=[
            pltpu.VMEM_SHARED((NP,), jnp.float32),
            pltpu.VMEM_SHARED((NP,), jnp.float32),
            pltpu.VMEM_SHARED((NP,), jnp.float32),
            pltpu.VMEM((S16_MAX, B), jnp.int32),
            pltpu.VMEM((B,), jnp.float32),
            pltpu.VMEM((ROWS_PER_CW,), jnp.float32),
            pltpu.VMEM((16, F), jnp.float32),
        ],
    )


def _edge_body(tab_s, tab_m, tab_l, src_s, dst_s, src_m, dst_m, src_l, dst_l,
               zrows, pout, acc, srcbuf, dstbuf, rows0, rows1, sem0, sem1):
    c = lax.axis_index("c")
    s = lax.axis_index("s")
    w = s * NC + c
    rbase = s * ROWS_PER_TILE
    rows = (rows0, rows1)
    sems = (sem0, sem1)

    for k, (tab, src3, dst3) in enumerate(
            ((tab_s, src_s, dst_s), (tab_m, src_m, dst_m), (tab_l, src_l, dst_l))):
        pltpu.sync_copy(zrows.at[pl.ds(rbase, ROWS_PER_TILE)],
                        acc.at[pl.ds(rbase, ROWS_PER_TILE)])
        plsc.subcore_barrier()

        def blk_body(bi, _, tab=tab, src3=src3, dst3=dst3):
            pltpu.sync_copy(src3.at[w, pl.ds(bi * G, G)], srcbuf)
            pltpu.sync_copy(dst3.at[w, pl.ds(bi * G, G)], dstbuf)
            cp = pltpu.async_copy(tab.at[srcbuf.at[0]], rows[0], sems[0])
            for j in range(G):
                if j + 1 < G:
                    cpn = pltpu.async_copy(tab.at[srcbuf.at[j + 1]],
                                           rows[(j + 1) % 2], sems[(j + 1) % 2])
                cp.wait()
                pltpu.sync_copy(rows[j % 2], acc.at[dstbuf.at[j]], add=True)
                if j + 1 < G:
                    cp = cpn
            return _

        lax.fori_loop(0, S32[k] // G, blk_body, 0)
        plsc.subcore_barrier()

        pltpu.sync_copy(acc.at[pl.ds(rbase, ROWS_PER_TILE)],
                        pout.at[c, k, pl.ds(rbase, ROWS_PER_TILE)])
        plsc.subcore_barrier()


@functools.cache
def _edge_call():
    return pl.kernel(
        _edge_body,
        out_type=jax.ShapeDtypeStruct((NC, 3, NP, F), jnp.float32),
        mesh=_mesh(),
        compiler_params=pltpu.CompilerParams(needs_layout_passes=False),
        scratch_types=[
            pltpu.VMEM_SHARED((NP, F), jnp.float32),
            pltpu.VMEM((G, B), jnp.int32),
            pltpu.VMEM((G, B), jnp.int32),
            pltpu.VMEM((B, F), jnp.float32),
            pltpu.VMEM((B, F), jnp.float32),
            pltpu.SemaphoreType.DMA,
            pltpu.SemaphoreType.DMA,
        ],
    )


R = 1024
GRID = NP // R


def _tc1_body(x_ref, ws_ref, wm_ref, wl_ref, deg_ref, hs_ref, hm_ref, hl_ref):
    xb = x_ref[...]
    for k, (w_ref, h_ref) in enumerate(
            ((ws_ref, hs_ref), (wm_ref, hm_ref), (wl_ref, hl_ref))):
        h = jnp.dot(xb, w_ref[...], preferred_element_type=jnp.float32)
        h_ref[...] = h * lax.rsqrt(deg_ref[k])


def _tc2_body(p_ref, hs_ref, hm_ref, hl_ref, deg_ref, b1_ref,
              ws_ref, wm_ref, wl_ref, os_ref, om_ref, ol_ref):
    for k, (h_ref, w_ref, o_ref) in enumerate(
            ((hs_ref, ws_ref, os_ref), (hm_ref, wm_ref, om_ref), (hl_ref, wl_ref, ol_ref))):
        d = lax.rsqrt(deg_ref[k])
        p = p_ref[0, k] + p_ref[1, k] + h_ref[...]
        z = jnp.maximum(d * p + b1_ref[k], 0.0)
        o_ref[...] = jnp.dot(z, w_ref[...], preferred_element_type=jnp.float32) * d


def _tc3_body(p_ref, hs_ref, hm_ref, hl_ref, deg_ref, b2_ref, wmix_ref, out_ref):
    acc = None
    for k, h_ref in enumerate((hs_ref, hm_ref, hl_ref)):
        o = lax.rsqrt(deg_ref[k]) * (p_ref[0, k] + p_ref[1, k] + h_ref[...]) + b2_ref[k]
        t = wmix_ref[k] * o
        acc = t if acc is None else acc + t
    out_ref[...] = acc


def _bs_rows():
    return pl.BlockSpec((R, F), lambda i: (i, 0))


def _bs_w():
    return pl.BlockSpec((F, F), lambda i: (0, 0))


def _bs_dinv():
    return pl.BlockSpec((3, R, F), lambda i: (0, i, 0))


def _bs_part():
    return pl.BlockSpec((NC, 3, R, F), lambda i: (0, 0, i, 0))


def _bs_bias():
    return pl.BlockSpec((3, 1, F), lambda i: (0, 0, 0))


_tc1_call = pl.pallas_call(
    _tc1_body,
    grid=(GRID,),
    in_specs=[_bs_rows(), _bs_w(), _bs_w(), _bs_w(), _bs_dinv()],
    out_specs=[_bs_rows()] * 3,
    out_shape=[jax.ShapeDtypeStruct((NP, F), jnp.float32)] * 3,
)

_tc2_call = pl.pallas_call(
    _tc2_body,
    grid=(GRID,),
    in_specs=[_bs_part(), _bs_rows(), _bs_rows(), _bs_rows(), _bs_dinv(),
              _bs_bias(), _bs_w(), _bs_w(), _bs_w()],
    out_specs=[_bs_rows()] * 3,
    out_shape=[jax.ShapeDtypeStruct((NP, F), jnp.float32)] * 3,
)

_tc3_call = pl.pallas_call(
    _tc3_body,
    grid=(GRID,),
    in_specs=[_bs_part(), _bs_rows(), _bs_rows(), _bs_rows(), _bs_dinv(),
              _bs_bias(), _bs_bias()],
    out_specs=_bs_rows(),
    out_shape=jax.ShapeDtypeStruct((NP, F), jnp.float32),
)


def _pad_edges(edge, k):
    e = E_RAW[k]
    pad = E_PAD[k] - e
    ar = lax.iota(jnp.int32, pad)
    src = jnp.concatenate([edge[0], ar % N])
    dst = jnp.concatenate([edge[1], N + (ar % 16)])
    return src, dst


def kernel(x, edge_small, edge_medium, edge_large, Ws1, bs1, Ws2, bs2,
           Wm1, bm1, Wm2, bm2, Wl1, bl1, Wl2, bl2, scale_weights):
    x_pad = jnp.pad(x, ((0, NP - N), (0, 0)))

    srcs, dsts = [], []
    for k, edge in enumerate((edge_small, edge_medium, edge_large)):
        src, dst = _pad_edges(edge, k)
        srcs.append(src)
        dsts.append(dst)

    deg3 = _deg_call()(dsts[0].reshape(NS, S16[0], B),
                      dsts[1].reshape(NS, S16[1], B),
                      dsts[2].reshape(NS, S16[2], B),
                      jnp.zeros((NP,), jnp.float32))

    h1s, h1m, h1l = _tc1_call(x_pad, Ws1, Wm1, Wl1, deg3)

    zrows = jnp.zeros((NP, F), jnp.float32)
    e32 = []
    for k in range(3):
        e32.append(srcs[k].reshape(NW, S32[k], B))
        e32.append(dsts[k].reshape(NW, S32[k], B))

    p1 = _edge_call()(h1s, h1m, h1l, *e32, zrows)

    b1 = jnp.stack([bs1, bm1, bl1])[:, None, :]
    h2s, h2m, h2l = _tc2_call(p1, h1s, h1m, h1l, deg3, b1, Ws2, Wm2, Wl2)

    p2 = _edge_call()(h2s, h2m, h2l, *e32, zrows)

    b2 = jnp.stack([bs2, bm2, bl2])[:, None, :]
    w = jax.nn.softmax(scale_weights)
    wmix = jnp.broadcast_to(w[:, None, None], (3, 1, F))
    out = _tc3_call(p2, h2s, h2m, h2l, deg3, b2, wmix)
    return out[:N]


# speedup vs baseline: 28.6042x; 1.0650x over previous
import functools

import jax
import jax.numpy as jnp
from jax import lax
from jax.experimental import pallas as pl
from jax.experimental.pallas import tpu as pltpu
from jax.experimental.pallas import tpu_sc as plsc

N = 10000
F = 128
NP = 10240
NC = 2
NS = 16
NW = NC * NS
B = 128

G = 8

BE = 64
GE = 16
NBUF = 4
DD = 3
SB = BE * F * 4

E_RAW = (160000, 320000, 640000)
E_PAD = tuple(-(-e // (NW * BE * GE)) * (NW * BE * GE) for e in E_RAW)
S32 = tuple(e // (NW * BE) for e in E_PAD)
S16 = tuple(e // (NS * B) for e in E_PAD)
S16_MAX = max(S16)

ROWS_PER_TILE = NP // NS
ROWS_PER_CW = NP // (NC * NS)

@functools.cache
def _mesh():
    return plsc.VectorSubcoreMesh(core_axis_name="c", subcore_axis_name="s")


def _deg_body(dst_s, dst_m, dst_l, zflat, deg_out,
              acc0, acc1, acc2, idxbuf, ones_v, degbuf, rowbuf):
    c = lax.axis_index("c")
    s = lax.axis_index("s")
    accs = (acc0, acc1, acc2)

    for i in range(B // 16):
        ones_v[pl.ds(i * 16, 16)] = jnp.full((16,), 1.0, dtype=jnp.float32)

    for k in range(3):
        pltpu.sync_copy(zflat.at[pl.ds(s * ROWS_PER_TILE, ROWS_PER_TILE)],
                        accs[k].at[pl.ds(s * ROWS_PER_TILE, ROWS_PER_TILE)])
    plsc.subcore_barrier()

    for k, dst3 in enumerate((dst_s, dst_m, dst_l)):
        pltpu.sync_copy(dst3.at[s], idxbuf.at[pl.ds(0, S16[k])])
        acck = accs[k]

        def body(j, _, acck=acck):
            pltpu.sync_copy(ones_v, acck.at[idxbuf.at[j]], add=True)
            return _

        lax.fori_loop(0, S16[k], body, 0)
    plsc.subcore_barrier()

    nbase = (c * NS + s) * ROWS_PER_CW
    for k in range(3):
        pltpu.sync_copy(accs[k].at[pl.ds(nbase, ROWS_PER_CW)], degbuf)

        def rep_body(g, _, k=k, nbase=nbase):
            for i in range(16):
                idxv = lax.broadcast_in_dim(g * 16 + i, (16,), ())
                row = plsc.load_gather(degbuf, [idxv]) + 1.0
                for cc in range(F // 16):
                    rowbuf[i, pl.ds(cc * 16, 16)] = row
            pltpu.sync_copy(rowbuf, deg_out.at[k, pl.ds(nbase + g * 16, 16)])
            return _

        lax.fori_loop(0, ROWS_PER_CW // 16, rep_body, 0)


@functools.cache
def _deg_call():
    return pl.kernel(
        _deg_body,
        out_type=jax.ShapeDtypeStruct((3, NP, F), jnp.float32),
        mesh=_mesh(),
        compiler_params=pltpu.CompilerParams(needs_layout_passes=False),
        scratch_types=[
            pltpu.VMEM_SHARED((NP,), jnp.float32),
            pltpu.VMEM_SHARED((NP,), jnp.float32),
            pltpu.VMEM_SHARED((NP,), jnp.float32),
            pltpu.VMEM((S16_MAX, B), jnp.int32),
            pltpu.VMEM((B,), jnp.float32),
            pltpu.VMEM((ROWS_PER_CW,), jnp.float32),
            pltpu.VMEM((16, F), jnp.float32),
        ],
    )


def _edge_body(tab_s, tab_m, tab_l, src_s, dst_s, src_m, dst_m, src_l, dst_l,
               zrows, pout, acc, srcbuf, dstbuf,
               rows0, rows1, rows2, rows3,
               sg0, sg1, sg2, sg3):
    c = lax.axis_index("c")
    s = lax.axis_index("s")
    w = s * NC + c
    rbase = s * ROWS_PER_TILE
    rows = (rows0, rows1, rows2, rows3)
    sg = (sg0, sg1, sg2, sg3)

    for k, (tab, src3, dst3) in enumerate(
            ((tab_s, src_s, dst_s), (tab_m, src_m, dst_m), (tab_l, src_l, dst_l))):
        pltpu.sync_copy(zrows.at[pl.ds(rbase, ROWS_PER_TILE)],
                        acc.at[pl.ds(rbase, ROWS_PER_TILE)])
        plsc.subcore_barrier()

        def blk_body(bi, _, tab=tab, src3=src3, dst3=dst3):
            pltpu.sync_copy(src3.at[w, pl.ds(bi * GE, GE)], srcbuf)
            pltpu.sync_copy(dst3.at[w, pl.ds(bi * GE, GE)], dstbuf)
            descs = [None] * GE
            for j in range(GE):
                p = j % NBUF
                descs[j] = pltpu.async_copy(tab.at[srcbuf.at[j]], rows[p], sg[p])
                if j >= DD:
                    q = (j - DD) % NBUF
                    descs[j - DD].wait()
                    pltpu.sync_copy(rows[q], acc.at[dstbuf.at[j - DD]], add=True)
            for r in range(GE - DD, GE):
                q = r % NBUF
                descs[r].wait()
                pltpu.sync_copy(rows[q], acc.at[dstbuf.at[r]], add=True)
            return _

        lax.fori_loop(0, S32[k] // GE, blk_body, 0)
        plsc.subcore_barrier()

        pltpu.sync_copy(acc.at[pl.ds(rbase, ROWS_PER_TILE)],
                        pout.at[c, k, pl.ds(rbase, ROWS_PER_TILE)])
        plsc.subcore_barrier()


@functools.cache
def _edge_call():
    return pl.kernel(
        _edge_body,
        out_type=jax.ShapeDtypeStruct((NC, 3, NP, F), jnp.float32),
        mesh=_mesh(),
        compiler_params=pltpu.CompilerParams(needs_layout_passes=False),
        scratch_types=[
            pltpu.VMEM_SHARED((NP, F), jnp.float32),
            pltpu.VMEM((GE, BE), jnp.int32),
            pltpu.VMEM((GE, BE), jnp.int32),
            pltpu.VMEM((BE, F), jnp.float32),
            pltpu.VMEM((BE, F), jnp.float32),
            pltpu.VMEM((BE, F), jnp.float32),
            pltpu.VMEM((BE, F), jnp.float32),
            pltpu.SemaphoreType.DMA,
            pltpu.SemaphoreType.DMA,
            pltpu.SemaphoreType.DMA,
            pltpu.SemaphoreType.DMA,
        ],
    )


R = 1024
GRID = NP // R


def _tc1_body(x_ref, ws_ref, wm_ref, wl_ref, deg_ref, hs_ref, hm_ref, hl_ref):
    xb = x_ref[...]
    for k, (w_ref, h_ref) in enumerate(
            ((ws_ref, hs_ref), (wm_ref, hm_ref), (wl_ref, hl_ref))):
        h = jnp.dot(xb, w_ref[...], preferred_element_type=jnp.float32)
        h_ref[...] = h * lax.rsqrt(deg_ref[k])


def _tc2_body(p_ref, hs_ref, hm_ref, hl_ref, deg_ref, b1_ref,
              ws_ref, wm_ref, wl_ref, os_ref, om_ref, ol_ref):
    for k, (h_ref, w_ref, o_ref) in enumerate(
            ((hs_ref, ws_ref, os_ref), (hm_ref, wm_ref, om_ref), (hl_ref, wl_ref, ol_ref))):
        d = lax.rsqrt(deg_ref[k])
        p = p_ref[0, k] + p_ref[1, k] + h_ref[...]
        z = jnp.maximum(d * p + b1_ref[k], 0.0)
        o_ref[...] = jnp.dot(z, w_ref[...], preferred_element_type=jnp.float32) * d


def _tc3_body(p_ref, hs_ref, hm_ref, hl_ref, deg_ref, b2_ref, wmix_ref, out_ref):
    acc = None
    for k, h_ref in enumerate((hs_ref, hm_ref, hl_ref)):
        o = lax.rsqrt(deg_ref[k]) * (p_ref[0, k] + p_ref[1, k] + h_ref[...]) + b2_ref[k]
        t = wmix_ref[k] * o
        acc = t if acc is None else acc + t
    out_ref[...] = acc


def _bs_rows():
    return pl.BlockSpec((R, F), lambda i: (i, 0))


def _bs_w():
    return pl.BlockSpec((F, F), lambda i: (0, 0))


def _bs_dinv():
    return pl.BlockSpec((3, R, F), lambda i: (0, i, 0))


def _bs_part():
    return pl.BlockSpec((NC, 3, R, F), lambda i: (0, 0, i, 0))


def _bs_bias():
    return pl.BlockSpec((3, 1, F), lambda i: (0, 0, 0))


_tc1_call = pl.pallas_call(
    _tc1_body,
    grid=(GRID,),
    in_specs=[_bs_rows(), _bs_w(), _bs_w(), _bs_w(), _bs_dinv()],
    out_specs=[_bs_rows()] * 3,
    out_shape=[jax.ShapeDtypeStruct((NP, F), jnp.float32)] * 3,
)

_tc2_call = pl.pallas_call(
    _tc2_body,
    grid=(GRID,),
    in_specs=[_bs_part(), _bs_rows(), _bs_rows(), _bs_rows(), _bs_dinv(),
              _bs_bias(), _bs_w(), _bs_w(), _bs_w()],
    out_specs=[_bs_rows()] * 3,
    out_shape=[jax.ShapeDtypeStruct((NP, F), jnp.float32)] * 3,
)

_tc3_call = pl.pallas_call(
    _tc3_body,
    grid=(GRID,),
    in_specs=[_bs_part(), _bs_rows(), _bs_rows(), _bs_rows(), _bs_dinv(),
              _bs_bias(), _bs_bias()],
    out_specs=_bs_rows(),
    out_shape=jax.ShapeDtypeStruct((NP, F), jnp.float32),
)


def _pad_edges(edge, k):
    e = E_RAW[k]
    pad = E_PAD[k] - e
    ar = lax.iota(jnp.int32, pad)
    src = jnp.concatenate([edge[0], ar % N])
    dst = jnp.concatenate([edge[1], N + (ar % 16)])
    return src, dst


def kernel(x, edge_small, edge_medium, edge_large, Ws1, bs1, Ws2, bs2,
           Wm1, bm1, Wm2, bm2, Wl1, bl1, Wl2, bl2, scale_weights):
    x_pad = jnp.pad(x, ((0, NP - N), (0, 0)))

    srcs, dsts = [], []
    for k, edge in enumerate((edge_small, edge_medium, edge_large)):
        src, dst = _pad_edges(edge, k)
        srcs.append(src)
        dsts.append(dst)

    deg3 = _deg_call()(dsts[0].reshape(NS, S16[0], B),
                      dsts[1].reshape(NS, S16[1], B),
                      dsts[2].reshape(NS, S16[2], B),
                      jnp.zeros((NP,), jnp.float32))

    h1s, h1m, h1l = _tc1_call(x_pad, Ws1, Wm1, Wl1, deg3)

    zrows = jnp.zeros((NP, F), jnp.float32)
    e32 = []
    for k in range(3):
        e32.append(srcs[k].reshape(NW, S32[k], BE))
        e32.append(dsts[k].reshape(NW, S32[k], BE))

    p1 = _edge_call()(h1s, h1m, h1l, *e32, zrows)

    b1 = jnp.stack([bs1, bm1, bl1])[:, None, :]
    h2s, h2m, h2l = _tc2_call(p1, h1s, h1m, h1l, deg3, b1, Ws2, Wm2, Wl2)

    p2 = _edge_call()(h2s, h2m, h2l, *e32, zrows)

    b2 = jnp.stack([bs2, bm2, bl2])[:, None, :]
    w = jax.nn.softmax(scale_weights)
    wmix = jnp.broadcast_to(w[:, None, None], (3, 1, F))
    out = _tc3_call(p2, h2s, h2m, h2l, deg3, b2, wmix)
    return out[:N]


# speedup vs baseline: 28.7119x; 1.0038x over previous
import functools

import jax
import jax.numpy as jnp
from jax import lax
from jax.experimental import pallas as pl
from jax.experimental.pallas import tpu as pltpu
from jax.experimental.pallas import tpu_sc as plsc

N = 10000
F = 128
NP = 10240
NC = 2
NS = 16
NW = NC * NS
B = 128

G = 8

BE = 64
GE = 16
NBUF = 4
DD = 3
SB = BE * F * 4

E_RAW = (160000, 320000, 640000)
E_PAD = tuple(-(-e // (NW * BE * GE)) * (NW * BE * GE) for e in E_RAW)
S32 = tuple(e // (NW * BE) for e in E_PAD)
S16 = tuple(e // (NS * B) for e in E_PAD)
S16_MAX = max(S16)

ROWS_PER_TILE = NP // NS
ROWS_PER_CW = NP // (NC * NS)

@functools.cache
def _mesh():
    return plsc.VectorSubcoreMesh(core_axis_name="c", subcore_axis_name="s")


def _deg_body(dst_s, dst_m, dst_l, zflat, deg_out,
              acc0, acc1, acc2, idxbuf, ones_v, degbuf, rowbuf):
    c = lax.axis_index("c")
    s = lax.axis_index("s")
    accs = (acc0, acc1, acc2)

    for i in range(B // 16):
        ones_v[pl.ds(i * 16, 16)] = jnp.full((16,), 1.0, dtype=jnp.float32)

    for k in range(3):
        pltpu.sync_copy(zflat.at[pl.ds(s * ROWS_PER_TILE, ROWS_PER_TILE)],
                        accs[k].at[pl.ds(s * ROWS_PER_TILE, ROWS_PER_TILE)])
    plsc.subcore_barrier()

    for k, dst3 in enumerate((dst_s, dst_m, dst_l)):
        pltpu.sync_copy(dst3.at[s], idxbuf.at[pl.ds(0, S16[k])])
        acck = accs[k]

        def body(j, _, acck=acck):
            pltpu.sync_copy(ones_v, acck.at[idxbuf.at[j]], add=True)
            return _

        lax.fori_loop(0, S16[k], body, 0)
    plsc.subcore_barrier()

    nbase = (c * NS + s) * ROWS_PER_CW
    for k in range(3):
        pltpu.sync_copy(accs[k].at[pl.ds(nbase, ROWS_PER_CW)], degbuf)

        def rep_body(g, _, k=k, nbase=nbase):
            for i in range(16):
                idxv = lax.broadcast_in_dim(g * 16 + i, (16,), ())
                row = plsc.load_gather(degbuf, [idxv]) + 1.0
                for cc in range(F // 16):
                    rowbuf[i, pl.ds(cc * 16, 16)] = row
            pltpu.sync_copy(rowbuf, deg_out.at[k, pl.ds(nbase + g * 16, 16)])
            return _

        lax.fori_loop(0, ROWS_PER_CW // 16, rep_body, 0)


@functools.cache
def _deg_call():
    return pl.kernel(
        _deg_body,
        out_type=jax.ShapeDtypeStruct((3, NP, F), jnp.float32),
        mesh=_mesh(),
        compiler_params=pltpu.CompilerParams(needs_layout_passes=False),
        scratch_types=[
            pltpu.VMEM_SHARED((NP,), jnp.float32),
            pltpu.VMEM_SHARED((NP,), jnp.float32),
            pltpu.VMEM_SHARED((NP,), jnp.float32),
            pltpu.VMEM((S16_MAX, B), jnp.int32),
            pltpu.VMEM((B,), jnp.float32),
            pltpu.VMEM((ROWS_PER_CW,), jnp.float32),
            pltpu.VMEM((16, F), jnp.float32),
        ],
    )


def _edge_body(tab_s, tab_m, tab_l, src_s, dst_s, src_m, dst_m, src_l, dst_l,
               zrows, pout, acc, srcbuf, dstbuf,
               rows0, rows1, rows2, rows3,
               sg0, sg1, sg2, sg3, ss0, ss1, ss2, ss3):
    c = lax.axis_index("c")
    s = lax.axis_index("s")
    w = s * NC + c
    rbase = s * ROWS_PER_TILE
    rows = (rows0, rows1, rows2, rows3)
    sg = (sg0, sg1, sg2, sg3)
    ss = (ss0, ss1, ss2, ss3)

    for k, (tab, src3, dst3) in enumerate(
            ((tab_s, src_s, dst_s), (tab_m, src_m, dst_m), (tab_l, src_l, dst_l))):
        pltpu.sync_copy(zrows.at[pl.ds(rbase, ROWS_PER_TILE)],
                        acc.at[pl.ds(rbase, ROWS_PER_TILE)])
        plsc.subcore_barrier()

        def blk_body(bi, _, tab=tab, src3=src3, dst3=dst3):
            pltpu.sync_copy(src3.at[w, pl.ds(bi * GE, GE)], srcbuf)
            pltpu.sync_copy(dst3.at[w, pl.ds(bi * GE, GE)], dstbuf)
            gd = [None] * GE
            sd = [None] * GE
            for j in range(GE):
                p = j % NBUF
                if j >= NBUF:
                    sd[j - NBUF].wait()
                gd[j] = pltpu.async_copy(tab.at[srcbuf.at[j]], rows[p], sg[p])
                if j >= DD:
                    q = (j - DD) % NBUF
                    gd[j - DD].wait()
                    sd[j - DD] = pltpu.async_copy(
                        rows[q], acc.at[dstbuf.at[j - DD]], ss[q], add=True)
            for r in range(GE - DD, GE):
                q = r % NBUF
                gd[r].wait()
                sd[r] = pltpu.async_copy(rows[q], acc.at[dstbuf.at[r]],
                                         ss[q], add=True)
            for r in range(GE - NBUF, GE):
                sd[r].wait()
            return _

        lax.fori_loop(0, S32[k] // GE, blk_body, 0)
        plsc.subcore_barrier()

        pltpu.sync_copy(acc.at[pl.ds(rbase, ROWS_PER_TILE)],
                        pout.at[c, k, pl.ds(rbase, ROWS_PER_TILE)])
        plsc.subcore_barrier()


@functools.cache
def _edge_call():
    return pl.kernel(
        _edge_body,
        out_type=jax.ShapeDtypeStruct((NC, 3, NP, F), jnp.float32),
        mesh=_mesh(),
        compiler_params=pltpu.CompilerParams(needs_layout_passes=False),
        scratch_types=[
            pltpu.VMEM_SHARED((NP, F), jnp.float32),
            pltpu.VMEM((GE, BE), jnp.int32),
            pltpu.VMEM((GE, BE), jnp.int32),
            pltpu.VMEM((BE, F), jnp.float32),
            pltpu.VMEM((BE, F), jnp.float32),
            pltpu.VMEM((BE, F), jnp.float32),
            pltpu.VMEM((BE, F), jnp.float32),
            pltpu.SemaphoreType.DMA,
            pltpu.SemaphoreType.DMA,
            pltpu.SemaphoreType.DMA,
            pltpu.SemaphoreType.DMA,
            pltpu.SemaphoreType.DMA,
            pltpu.SemaphoreType.DMA,
            pltpu.SemaphoreType.DMA,
            pltpu.SemaphoreType.DMA,
        ],
    )


R = 1024
GRID = NP // R


def _tc1_body(x_ref, ws_ref, wm_ref, wl_ref, deg_ref, hs_ref, hm_ref, hl_ref):
    xb = x_ref[...]
    for k, (w_ref, h_ref) in enumerate(
            ((ws_ref, hs_ref), (wm_ref, hm_ref), (wl_ref, hl_ref))):
        h = jnp.dot(xb, w_ref[...], preferred_element_type=jnp.float32)
        h_ref[...] = h * lax.rsqrt(deg_ref[k])


def _tc2_body(p_ref, hs_ref, hm_ref, hl_ref, deg_ref, b1_ref,
              ws_ref, wm_ref, wl_ref, os_ref, om_ref, ol_ref):
    for k, (h_ref, w_ref, o_ref) in enumerate(
            ((hs_ref, ws_ref, os_ref), (hm_ref, wm_ref, om_ref), (hl_ref, wl_ref, ol_ref))):
        d = lax.rsqrt(deg_ref[k])
        p = p_ref[0, k] + p_ref[1, k] + h_ref[...]
        z = jnp.maximum(d * p + b1_ref[k], 0.0)
        o_ref[...] = jnp.dot(z, w_ref[...], preferred_element_type=jnp.float32) * d


def _tc3_body(p_ref, hs_ref, hm_ref, hl_ref, deg_ref, b2_ref, wmix_ref, out_ref):
    acc = None
    for k, h_ref in enumerate((hs_ref, hm_ref, hl_ref)):
        o = lax.rsqrt(deg_ref[k]) * (p_ref[0, k] + p_ref[1, k] + h_ref[...]) + b2_ref[k]
        t = wmix_ref[k] * o
        acc = t if acc is None else acc + t
    out_ref[...] = acc


def _bs_rows():
    return pl.BlockSpec((R, F), lambda i: (i, 0))


def _bs_w():
    return pl.BlockSpec((F, F), lambda i: (0, 0))


def _bs_dinv():
    return pl.BlockSpec((3, R, F), lambda i: (0, i, 0))


def _bs_part():
    return pl.BlockSpec((NC, 3, R, F), lambda i: (0, 0, i, 0))


def _bs_bias():
    return pl.BlockSpec((3, 1, F), lambda i: (0, 0, 0))


_tc1_call = pl.pallas_call(
    _tc1_body,
    grid=(GRID,),
    in_specs=[_bs_rows(), _bs_w(), _bs_w(), _bs_w(), _bs_dinv()],
    out_specs=[_bs_rows()] * 3,
    out_shape=[jax.ShapeDtypeStruct((NP, F), jnp.float32)] * 3,
)

_tc2_call = pl.pallas_call(
    _tc2_body,
    grid=(GRID,),
    in_specs=[_bs_part(), _bs_rows(), _bs_rows(), _bs_rows(), _bs_dinv(),
              _bs_bias(), _bs_w(), _bs_w(), _bs_w()],
    out_specs=[_bs_rows()] * 3,
    out_shape=[jax.ShapeDtypeStruct((NP, F), jnp.float32)] * 3,
)

_tc3_call = pl.pallas_call(
    _tc3_body,
    grid=(GRID,),
    in_specs=[_bs_part(), _bs_rows(), _bs_rows(), _bs_rows(), _bs_dinv(),
              _bs_bias(), _bs_bias()],
    out_specs=_bs_rows(),
    out_shape=jax.ShapeDtypeStruct((NP, F), jnp.float32),
)


def _pad_edges(edge, k):
    e = E_RAW[k]
    pad = E_PAD[k] - e
    ar = lax.iota(jnp.int32, pad)
    src = jnp.concatenate([edge[0], ar % N])
    dst = jnp.concatenate([edge[1], N + (ar % 16)])
    return src, dst


def kernel(x, edge_small, edge_medium, edge_large, Ws1, bs1, Ws2, bs2,
           Wm1, bm1, Wm2, bm2, Wl1, bl1, Wl2, bl2, scale_weights):
    x_pad = jnp.pad(x, ((0, NP - N), (0, 0)))

    srcs, dsts = [], []
    for k, edge in enumerate((edge_small, edge_medium, edge_large)):
        src, dst = _pad_edges(edge, k)
        srcs.append(src)
        dsts.append(dst)

    deg3 = _deg_call()(dsts[0].reshape(NS, S16[0], B),
                      dsts[1].reshape(NS, S16[1], B),
                      dsts[2].reshape(NS, S16[2], B),
                      jnp.zeros((NP,), jnp.float32))

    h1s, h1m, h1l = _tc1_call(x_pad, Ws1, Wm1, Wl1, deg3)

    zrows = jnp.zeros((NP, F), jnp.float32)
    e32 = []
    for k in range(3):
        e32.append(srcs[k].reshape(NW, S32[k], BE))
        e32.append(dsts[k].reshape(NW, S32[k], BE))

    p1 = _edge_call()(h1s, h1m, h1l, *e32, zrows)

    b1 = jnp.stack([bs1, bm1, bl1])[:, None, :]
    h2s, h2m, h2l = _tc2_call(p1, h1s, h1m, h1l, deg3, b1, Ws2, Wm2, Wl2)

    p2 = _edge_call()(h2s, h2m, h2l, *e32, zrows)

    b2 = jnp.stack([bs2, bm2, bl2])[:, None, :]
    w = jax.nn.softmax(scale_weights)
    wmix = jnp.broadcast_to(w[:, None, None], (3, 1, F))
    out = _tc3_call(p2, h2s, h2m, h2l, deg3, b2, wmix)
    return out[:N]


# speedup vs baseline: 29.8263x; 1.0388x over previous
import functools

import jax
import jax.numpy as jnp
from jax import lax
from jax.experimental import pallas as pl
from jax.experimental.pallas import tpu as pltpu
from jax.experimental.pallas import tpu_sc as plsc

N = 10000
F = 128
NP = 10240
NC = 2
NS = 16
NW = NC * NS
B = 128

G = 8

BE = 64
GE = 16
NBUF = 4
DD = 3
SB = BE * F * 4

E_RAW = (160000, 320000, 640000)
E_PAD = tuple(-(-e // (NW * BE * GE)) * (NW * BE * GE) for e in E_RAW)
S32 = tuple(e // (NW * BE) for e in E_PAD)
S16 = tuple(e // (NS * B) for e in E_PAD)
S16_MAX = max(S16)

ROWS_PER_TILE = NP // NS
ROWS_PER_CW = NP // (NC * NS)

@functools.cache
def _mesh():
    return plsc.VectorSubcoreMesh(core_axis_name="c", subcore_axis_name="s")


def _deg_body(dst_s, dst_m, dst_l, zflat, deg_out,
              acc0, acc1, acc2, idxbuf, ones_v, degbuf, rowbuf0, rowbuf1,
              sd0, sd1, sd2, sd3, sr0, sr1):
    c = lax.axis_index("c")
    s = lax.axis_index("s")
    accs = (acc0, acc1, acc2)
    sd = (sd0, sd1, sd2, sd3)

    for i in range(B // 16):
        ones_v[pl.ds(i * 16, 16)] = jnp.full((16,), 1.0, dtype=jnp.float32)

    for k in range(3):
        pltpu.sync_copy(zflat.at[pl.ds(s * ROWS_PER_TILE, ROWS_PER_TILE)],
                        accs[k].at[pl.ds(s * ROWS_PER_TILE, ROWS_PER_TILE)])
    plsc.subcore_barrier()

    for k, dst3 in enumerate((dst_s, dst_m, dst_l)):
        pltpu.sync_copy(dst3.at[s], idxbuf.at[pl.ds(0, S16[k])])
        acck = accs[k]

        def body(bi, _, acck=acck):
            ds_ = [None] * 4
            for t in range(4):
                ds_[t] = pltpu.async_copy(
                    ones_v, acck.at[idxbuf.at[bi * 4 + t]], sd[t], add=True)
            for t in range(4):
                ds_[t].wait()
            return _

        lax.fori_loop(0, S16[k] // 4, body, 0)
    plsc.subcore_barrier()

    nbase = (c * NS + s) * ROWS_PER_CW
    for k in range(3):
        pltpu.sync_copy(accs[k].at[pl.ds(nbase, ROWS_PER_CW)], degbuf)

        def rep_body(g, _, k=k, nbase=nbase):
            cps = [None, None]
            for h, rowbuf in enumerate((rowbuf0, rowbuf1)):
                g32 = g * 64 + h * 32
                for i in range(32):
                    idxv = lax.broadcast_in_dim(g32 + i, (16,), ())
                    row = plsc.load_gather(degbuf, [idxv]) + 1.0
                    for cc in range(F // 16):
                        rowbuf[i, pl.ds(cc * 16, 16)] = row
                cps[h] = pltpu.async_copy(
                    rowbuf, deg_out.at[k, pl.ds(nbase + g32, 32)],
                    (sr0, sr1)[h])
            cps[0].wait()
            cps[1].wait()
            return _

        lax.fori_loop(0, ROWS_PER_CW // 64, rep_body, 0)


@functools.cache
def _deg_call():
    return pl.kernel(
        _deg_body,
        out_type=jax.ShapeDtypeStruct((3, NP, F), jnp.float32),
        mesh=_mesh(),
        compiler_params=pltpu.CompilerParams(needs_layout_passes=False),
        scratch_types=[
            pltpu.VMEM_SHARED((NP,), jnp.float32),
            pltpu.VMEM_SHARED((NP,), jnp.float32),
            pltpu.VMEM_SHARED((NP,), jnp.float32),
            pltpu.VMEM((S16_MAX, B), jnp.int32),
            pltpu.VMEM((B,), jnp.float32),
            pltpu.VMEM((ROWS_PER_CW,), jnp.float32),
            pltpu.VMEM((32, F), jnp.float32),
            pltpu.VMEM((32, F), jnp.float32),
            pltpu.SemaphoreType.DMA,
            pltpu.SemaphoreType.DMA,
            pltpu.SemaphoreType.DMA,
            pltpu.SemaphoreType.DMA,
            pltpu.SemaphoreType.DMA,
            pltpu.SemaphoreType.DMA,
        ],
    )


def _edge_body(tab_s, tab_m, tab_l, src_s, dst_s, src_m, dst_m, src_l, dst_l,
               zrows, pout, acc, srcbuf, dstbuf,
               rows0, rows1, rows2, rows3,
               sg0, sg1, sg2, sg3, ss0, ss1, ss2, ss3):
    c = lax.axis_index("c")
    s = lax.axis_index("s")
    w = s * NC + c
    rbase = s * ROWS_PER_TILE
    rows = (rows0, rows1, rows2, rows3)
    sg = (sg0, sg1, sg2, sg3)
    ss = (ss0, ss1, ss2, ss3)

    for k, (tab, src3, dst3) in enumerate(
            ((tab_s, src_s, dst_s), (tab_m, src_m, dst_m), (tab_l, src_l, dst_l))):
        @pl.when(c == 0)
        def _(tab=tab):
            pltpu.sync_copy(tab.at[pl.ds(rbase, ROWS_PER_TILE)],
                            acc.at[pl.ds(rbase, ROWS_PER_TILE)])

        @pl.when(c != 0)
        def _():
            pltpu.sync_copy(zrows.at[pl.ds(rbase, ROWS_PER_TILE)],
                            acc.at[pl.ds(rbase, ROWS_PER_TILE)])

        plsc.subcore_barrier()

        def blk_body(bi, _, tab=tab, src3=src3, dst3=dst3):
            pltpu.sync_copy(src3.at[w, pl.ds(bi * GE, GE)], srcbuf)
            pltpu.sync_copy(dst3.at[w, pl.ds(bi * GE, GE)], dstbuf)
            gd = [None] * GE
            sd = [None] * GE
            for j in range(GE):
                p = j % NBUF
                if j >= NBUF:
                    sd[j - NBUF].wait()
                gd[j] = pltpu.async_copy(tab.at[srcbuf.at[j]], rows[p], sg[p])
                if j >= DD:
                    q = (j - DD) % NBUF
                    gd[j - DD].wait()
                    sd[j - DD] = pltpu.async_copy(
                        rows[q], acc.at[dstbuf.at[j - DD]], ss[q], add=True)
            for r in range(GE - DD, GE):
                q = r % NBUF
                gd[r].wait()
                sd[r] = pltpu.async_copy(rows[q], acc.at[dstbuf.at[r]],
                                         ss[q], add=True)
            for r in range(GE - NBUF, GE):
                sd[r].wait()
            return _

        lax.fori_loop(0, S32[k] // GE, blk_body, 0)
        plsc.subcore_barrier()

        pltpu.sync_copy(acc.at[pl.ds(rbase, ROWS_PER_TILE)],
                        pout.at[c, k, pl.ds(rbase, ROWS_PER_TILE)])
        plsc.subcore_barrier()


@functools.cache
def _edge_call():
    return pl.kernel(
        _edge_body,
        out_type=jax.ShapeDtypeStruct((NC, 3, NP, F), jnp.float32),
        mesh=_mesh(),
        compiler_params=pltpu.CompilerParams(needs_layout_passes=False),
        scratch_types=[
            pltpu.VMEM_SHARED((NP, F), jnp.float32),
            pltpu.VMEM((GE, BE), jnp.int32),
            pltpu.VMEM((GE, BE), jnp.int32),
            pltpu.VMEM((BE, F), jnp.float32),
            pltpu.VMEM((BE, F), jnp.float32),
            pltpu.VMEM((BE, F), jnp.float32),
            pltpu.VMEM((BE, F), jnp.float32),
            pltpu.SemaphoreType.DMA,
            pltpu.SemaphoreType.DMA,
            pltpu.SemaphoreType.DMA,
            pltpu.SemaphoreType.DMA,
            pltpu.SemaphoreType.DMA,
            pltpu.SemaphoreType.DMA,
            pltpu.SemaphoreType.DMA,
            pltpu.SemaphoreType.DMA,
        ],
    )


R = 1024
GRID = NP // R


def _tc1_body(x_ref, ws_ref, wm_ref, wl_ref, deg_ref, hs_ref, hm_ref, hl_ref):
    xb = x_ref[...]
    for k, (w_ref, h_ref) in enumerate(
            ((ws_ref, hs_ref), (wm_ref, hm_ref), (wl_ref, hl_ref))):
        h = jnp.dot(xb, w_ref[...], preferred_element_type=jnp.float32)
        h_ref[...] = h * lax.rsqrt(deg_ref[k])


def _tc2_body(p_ref, deg_ref, b1_ref,
              ws_ref, wm_ref, wl_ref, os_ref, om_ref, ol_ref):
    for k, (w_ref, o_ref) in enumerate(
            ((ws_ref, os_ref), (wm_ref, om_ref), (wl_ref, ol_ref))):
        d = lax.rsqrt(deg_ref[k])
        p = p_ref[0, k] + p_ref[1, k]
        z = jnp.maximum(d * p + b1_ref[k], 0.0)
        o_ref[...] = jnp.dot(z, w_ref[...], preferred_element_type=jnp.float32) * d


def _tc3_body(p_ref, deg_ref, b2_ref, wmix_ref, out_ref):
    acc = None
    for k in range(3):
        o = lax.rsqrt(deg_ref[k]) * (p_ref[0, k] + p_ref[1, k]) + b2_ref[k]
        t = wmix_ref[k] * o
        acc = t if acc is None else acc + t
    out_ref[...] = acc


def _bs_rows():
    return pl.BlockSpec((R, F), lambda i: (i, 0))


def _bs_w():
    return pl.BlockSpec((F, F), lambda i: (0, 0))


def _bs_dinv():
    return pl.BlockSpec((3, R, F), lambda i: (0, i, 0))


def _bs_part():
    return pl.BlockSpec((NC, 3, R, F), lambda i: (0, 0, i, 0))


def _bs_bias():
    return pl.BlockSpec((3, 1, F), lambda i: (0, 0, 0))


_tc1_call = pl.pallas_call(
    _tc1_body,
    grid=(GRID,),
    in_specs=[_bs_rows(), _bs_w(), _bs_w(), _bs_w(), _bs_dinv()],
    out_specs=[_bs_rows()] * 3,
    out_shape=[jax.ShapeDtypeStruct((NP, F), jnp.float32)] * 3,
)

_tc2_call = pl.pallas_call(
    _tc2_body,
    grid=(GRID,),
    in_specs=[_bs_part(), _bs_dinv(), _bs_bias(), _bs_w(), _bs_w(), _bs_w()],
    out_specs=[_bs_rows()] * 3,
    out_shape=[jax.ShapeDtypeStruct((NP, F), jnp.float32)] * 3,
)

_tc3_call = pl.pallas_call(
    _tc3_body,
    grid=(GRID,),
    in_specs=[_bs_part(), _bs_dinv(), _bs_bias(), _bs_bias()],
    out_specs=_bs_rows(),
    out_shape=jax.ShapeDtypeStruct((NP, F), jnp.float32),
)


def _pad_edges(edge, k):
    e = E_RAW[k]
    pad = E_PAD[k] - e
    ar = lax.iota(jnp.int32, pad)
    src = jnp.concatenate([edge[0], ar % N])
    dst = jnp.concatenate([edge[1], N + (ar % 16)])
    return src, dst


def kernel(x, edge_small, edge_medium, edge_large, Ws1, bs1, Ws2, bs2,
           Wm1, bm1, Wm2, bm2, Wl1, bl1, Wl2, bl2, scale_weights):
    x_pad = jnp.pad(x, ((0, NP - N), (0, 0)))

    srcs, dsts = [], []
    for k, edge in enumerate((edge_small, edge_medium, edge_large)):
        src, dst = _pad_edges(edge, k)
        srcs.append(src)
        dsts.append(dst)

    deg3 = _deg_call()(dsts[0].reshape(NS, S16[0], B),
                      dsts[1].reshape(NS, S16[1], B),
                      dsts[2].reshape(NS, S16[2], B),
                      jnp.zeros((NP,), jnp.float32))

    h1s, h1m, h1l = _tc1_call(x_pad, Ws1, Wm1, Wl1, deg3)

    zrows = jnp.zeros((NP, F), jnp.float32)
    e32 = []
    for k in range(3):
        e32.append(srcs[k].reshape(NW, S32[k], BE))
        e32.append(dsts[k].reshape(NW, S32[k], BE))

    p1 = _edge_call()(h1s, h1m, h1l, *e32, zrows)

    b1 = jnp.stack([bs1, bm1, bl1])[:, None, :]
    h2s, h2m, h2l = _tc2_call(p1, deg3, b1, Ws2, Wm2, Wl2)

    p2 = _edge_call()(h2s, h2m, h2l, *e32, zrows)

    b2 = jnp.stack([bs2, bm2, bl2])[:, None, :]
    w = jax.nn.softmax(scale_weights)
    wmix = jnp.broadcast_to(w[:, None, None], (3, 1, F))
    out = _tc3_call(p2, deg3, b2, wmix)
    return out[:N]


# speedup vs baseline: 29.8574x; 1.0010x over previous
import functools

import jax
import jax.numpy as jnp
from jax import lax
from jax.experimental import pallas as pl
from jax.experimental.pallas import tpu as pltpu
from jax.experimental.pallas import tpu_sc as plsc

N = 10000
F = 128
NP = 10240
NC = 2
NS = 16
NW = NC * NS
B = 128

BE = 64
GE = 16
NBUF = 4
DD = 3

E_RAW = (160000, 320000, 640000)
E_PAD = tuple(-(-e // (NW * BE * GE)) * (NW * BE * GE) for e in E_RAW)
S32 = tuple(e // (NW * BE) for e in E_PAD)
S16 = tuple(e // (NS * B) for e in E_PAD)
S16_MAX = max(S16)

ROWS_PER_TILE = NP // NS
ROWS_PER_CW = NP // (NC * NS)

@functools.cache
def _mesh():
    return plsc.VectorSubcoreMesh(core_axis_name="c", subcore_axis_name="s")


def _deg_body(dst_s, dst_m, dst_l, zflat, deg_out,
              acc0, acc1, acc2, idxbuf, ones_v, degbuf, rowbuf0, rowbuf1,
              sd0, sd1, sd2, sd3, sr0, sr1):
    c = lax.axis_index("c")
    s = lax.axis_index("s")
    accs = (acc0, acc1, acc2)
    sd = (sd0, sd1, sd2, sd3)

    for i in range(B // 16):
        ones_v[pl.ds(i * 16, 16)] = jnp.full((16,), 1.0, dtype=jnp.float32)

    for k in range(3):
        pltpu.sync_copy(zflat.at[pl.ds(s * ROWS_PER_TILE, ROWS_PER_TILE)],
                        accs[k].at[pl.ds(s * ROWS_PER_TILE, ROWS_PER_TILE)])
    plsc.subcore_barrier()

    for k, dst3 in enumerate((dst_s, dst_m, dst_l)):
        pltpu.sync_copy(dst3.at[s], idxbuf.at[pl.ds(0, S16[k])])
        acck = accs[k]

        def body(bi, _, acck=acck):
            ds_ = [None] * 4
            for t in range(4):
                ds_[t] = pltpu.async_copy(
                    ones_v, acck.at[idxbuf.at[bi * 4 + t]], sd[t], add=True)
            for t in range(4):
                ds_[t].wait()
            return _

        lax.fori_loop(0, S16[k] // 4, body, 0)
    plsc.subcore_barrier()

    nbase = (c * NS + s) * ROWS_PER_CW
    for k in range(3):
        pltpu.sync_copy(accs[k].at[pl.ds(nbase, ROWS_PER_CW)], degbuf)

        def rep_body(g, _, k=k, nbase=nbase):
            cps = [None, None]
            for h, rowbuf in enumerate((rowbuf0, rowbuf1)):
                g32 = g * 64 + h * 32
                for i in range(32):
                    idxv = lax.broadcast_in_dim(g32 + i, (16,), ())
                    row = plsc.load_gather(degbuf, [idxv]) + 1.0
                    for cc in range(F // 16):
                        rowbuf[i, pl.ds(cc * 16, 16)] = row
                cps[h] = pltpu.async_copy(
                    rowbuf, deg_out.at[k, pl.ds(nbase + g32, 32)],
                    (sr0, sr1)[h])
            cps[0].wait()
            cps[1].wait()
            return _

        lax.fori_loop(0, ROWS_PER_CW // 64, rep_body, 0)


@functools.cache
def _deg_call():
    return pl.kernel(
        _deg_body,
        out_type=jax.ShapeDtypeStruct((3, NP, F), jnp.float32),
        mesh=_mesh(),
        compiler_params=pltpu.CompilerParams(needs_layout_passes=False),
        scratch_types=[
            pltpu.VMEM_SHARED((NP,), jnp.float32),
            pltpu.VMEM_SHARED((NP,), jnp.float32),
            pltpu.VMEM_SHARED((NP,), jnp.float32),
            pltpu.VMEM((S16_MAX, B), jnp.int32),
            pltpu.VMEM((B,), jnp.float32),
            pltpu.VMEM((ROWS_PER_CW,), jnp.float32),
            pltpu.VMEM((32, F), jnp.float32),
            pltpu.VMEM((32, F), jnp.float32),
            pltpu.SemaphoreType.DMA,
            pltpu.SemaphoreType.DMA,
            pltpu.SemaphoreType.DMA,
            pltpu.SemaphoreType.DMA,
            pltpu.SemaphoreType.DMA,
            pltpu.SemaphoreType.DMA,
        ],
    )


def _edge_body(tab_s, tab_m, tab_l, src_s, dst_s, src_m, dst_m, src_l, dst_l,
               zrows, pout, acc, srcbuf, dstbuf,
               rows0, rows1, rows2, rows3,
               sg0, sg1, sg2, sg3, ss0, ss1, ss2, ss3):
    c = lax.axis_index("c")
    s = lax.axis_index("s")
    w = s * NC + c
    rbase = s * ROWS_PER_TILE
    rows = (rows0, rows1, rows2, rows3)
    sg = (sg0, sg1, sg2, sg3)
    ss = (ss0, ss1, ss2, ss3)

    for k, (tab, src3, dst3) in enumerate(
            ((tab_s, src_s, dst_s), (tab_m, src_m, dst_m), (tab_l, src_l, dst_l))):
        @pl.when(c == 0)
        def _(tab=tab):
            pltpu.sync_copy(tab.at[pl.ds(rbase, ROWS_PER_TILE)],
                            acc.at[pl.ds(rbase, ROWS_PER_TILE)])

        @pl.when(c != 0)
        def _():
            pltpu.sync_copy(zrows.at[pl.ds(rbase, ROWS_PER_TILE)],
                            acc.at[pl.ds(rbase, ROWS_PER_TILE)])

        plsc.subcore_barrier()

        def blk_body(bi, _, tab=tab, src3=src3, dst3=dst3):
            pltpu.sync_copy(src3.at[w, pl.ds(bi * GE, GE)], srcbuf)
            pltpu.sync_copy(dst3.at[w, pl.ds(bi * GE, GE)], dstbuf)
            gd = [None] * GE
            sd = [None] * GE
            for j in range(GE):
                p = j % NBUF
                if j >= NBUF:
                    sd[j - NBUF].wait()
                gd[j] = pltpu.async_copy(tab.at[srcbuf.at[j]], rows[p], sg[p])
                if j >= DD:
                    q = (j - DD) % NBUF
                    gd[j - DD].wait()
                    sd[j - DD] = pltpu.async_copy(
                        rows[q], acc.at[dstbuf.at[j - DD]], ss[q], add=True)
            for r in range(GE - DD, GE):
                q = r % NBUF
                gd[r].wait()
                sd[r] = pltpu.async_copy(rows[q], acc.at[dstbuf.at[r]],
                                         ss[q], add=True)
            for r in range(GE - NBUF, GE):
                sd[r].wait()
            return _

        lax.fori_loop(0, S32[k] // GE, blk_body, 0)
        plsc.subcore_barrier()

        pltpu.sync_copy(acc.at[pl.ds(rbase, ROWS_PER_TILE)],
                        pout.at[c, k, pl.ds(rbase, ROWS_PER_TILE)])
        plsc.subcore_barrier()


@functools.cache
def _edge_call():
    return pl.kernel(
        _edge_body,
        out_type=jax.ShapeDtypeStruct((NC, 3, NP, F), jnp.float32),
        mesh=_mesh(),
        compiler_params=pltpu.CompilerParams(needs_layout_passes=False),
        scratch_types=[
            pltpu.VMEM_SHARED((NP, F), jnp.float32),
            pltpu.VMEM((GE, BE), jnp.int32),
            pltpu.VMEM((GE, BE), jnp.int32),
            pltpu.VMEM((BE, F), jnp.float32),
            pltpu.VMEM((BE, F), jnp.float32),
            pltpu.VMEM((BE, F), jnp.float32),
            pltpu.VMEM((BE, F), jnp.float32),
            pltpu.SemaphoreType.DMA,
            pltpu.SemaphoreType.DMA,
            pltpu.SemaphoreType.DMA,
            pltpu.SemaphoreType.DMA,
            pltpu.SemaphoreType.DMA,
            pltpu.SemaphoreType.DMA,
            pltpu.SemaphoreType.DMA,
            pltpu.SemaphoreType.DMA,
        ],
    )


R = 1024
GRID = NP // R


def _tc1_body(x_ref, ws_ref, wm_ref, wl_ref, deg_ref, hs_ref, hm_ref, hl_ref):
    xb = x_ref[...]
    for k, (w_ref, h_ref) in enumerate(
            ((ws_ref, hs_ref), (wm_ref, hm_ref), (wl_ref, hl_ref))):
        h = jnp.dot(xb, w_ref[...], preferred_element_type=jnp.float32)
        h_ref[...] = h * lax.rsqrt(deg_ref[k])


def _tc2_body(p_ref, deg_ref, b1_ref,
              ws_ref, wm_ref, wl_ref, os_ref, om_ref, ol_ref):
    for k, (w_ref, o_ref) in enumerate(
            ((ws_ref, os_ref), (wm_ref, om_ref), (wl_ref, ol_ref))):
        d = lax.rsqrt(deg_ref[k])
        p = p_ref[0, k] + p_ref[1, k]
        z = jnp.maximum(d * p + b1_ref[k], 0.0)
        o_ref[...] = jnp.dot(z, w_ref[...], preferred_element_type=jnp.float32) * d


def _tc3_body(p_ref, deg_ref, b2_ref, wmix_ref, out_ref):
    acc = None
    for k in range(3):
        o = lax.rsqrt(deg_ref[k]) * (p_ref[0, k] + p_ref[1, k]) + b2_ref[k]
        t = wmix_ref[k] * o
        acc = t if acc is None else acc + t
    out_ref[...] = acc


def _bs_rows():
    return pl.BlockSpec((R, F), lambda i: (i, 0))


def _bs_w():
    return pl.BlockSpec((F, F), lambda i: (0, 0))


def _bs_dinv():
    return pl.BlockSpec((3, R, F), lambda i: (0, i, 0))


def _bs_part():
    return pl.BlockSpec((NC, 3, R, F), lambda i: (0, 0, i, 0))


def _bs_bias():
    return pl.BlockSpec((3, 1, F), lambda i: (0, 0, 0))


_tc1_call = pl.pallas_call(
    _tc1_body,
    grid=(GRID,),
    in_specs=[_bs_rows(), _bs_w(), _bs_w(), _bs_w(), _bs_dinv()],
    out_specs=[_bs_rows()] * 3,
    out_shape=[jax.ShapeDtypeStruct((NP, F), jnp.float32)] * 3,
)

_tc2_call = pl.pallas_call(
    _tc2_body,
    grid=(GRID,),
    in_specs=[_bs_part(), _bs_dinv(), _bs_bias(), _bs_w(), _bs_w(), _bs_w()],
    out_specs=[_bs_rows()] * 3,
    out_shape=[jax.ShapeDtypeStruct((NP, F), jnp.float32)] * 3,
)

_tc3_call = pl.pallas_call(
    _tc3_body,
    grid=(GRID,),
    in_specs=[_bs_part(), _bs_dinv(), _bs_bias(), _bs_bias()],
    out_specs=_bs_rows(),
    out_shape=jax.ShapeDtypeStruct((NP, F), jnp.float32),
)


def _pad_edges(edge, k):
    e = E_RAW[k]
    pad = E_PAD[k] - e
    ar = lax.iota(jnp.int32, pad)
    src = jnp.concatenate([edge[0], ar % N])
    dst = jnp.concatenate([edge[1], N + (ar % 16)])
    return src, dst


def kernel(x, edge_small, edge_medium, edge_large, Ws1, bs1, Ws2, bs2,
           Wm1, bm1, Wm2, bm2, Wl1, bl1, Wl2, bl2, scale_weights):
    x_pad = jnp.pad(x, ((0, NP - N), (0, 0)))

    srcs, dsts = [], []
    for k, edge in enumerate((edge_small, edge_medium, edge_large)):
        src, dst = _pad_edges(edge, k)
        srcs.append(src)
        dsts.append(dst)

    deg3 = _deg_call()(dsts[0].reshape(NS, S16[0], B),
                      dsts[1].reshape(NS, S16[1], B),
                      dsts[2].reshape(NS, S16[2], B),
                      jnp.zeros((NP,), jnp.float32))

    h1s, h1m, h1l = _tc1_call(x_pad, Ws1, Wm1, Wl1, deg3)

    zrows = jnp.zeros((NP, F), jnp.float32)
    e32 = []
    for k in range(3):
        e32.append(srcs[k].reshape(NW, S32[k], BE))
        e32.append(dsts[k].reshape(NW, S32[k], BE))

    p1 = _edge_call()(h1s, h1m, h1l, *e32, zrows)

    b1 = jnp.stack([bs1, bm1, bl1])[:, None, :]
    h2s, h2m, h2l = _tc2_call(p1, deg3, b1, Ws2, Wm2, Wl2)

    p2 = _edge_call()(h2s, h2m, h2l, *e32, zrows)

    b2 = jnp.stack([bs2, bm2, bl2])[:, None, :]
    w = jax.nn.softmax(scale_weights)
    wmix = jnp.broadcast_to(w[:, None, None], (3, 1, F))
    out = _tc3_call(p2, deg3, b2, wmix)
    return out[:N]
